# baseline (speedup 1.0000x reference)
import functools

import jax
import jax.numpy as jnp
from jax import lax
from jax.experimental import pallas as pl
from jax.experimental.pallas import tpu as pltpu
from jax.experimental.pallas import tpu_sc as plsc

_NC = 2
_NS = 16
_NW = _NC * _NS
_L = 16


def _sc_body(CH, BN, N, C, k, pts_hbm, idx_hbm, out_hbm,
             idx_raw, idx_b0, idx_b1, idx_b2, idx_b3, xv, rows, out_v, sem):
    KC = k * CH
    OUTW = 2 * C * k
    n_chunks = (BN // _NW) // CH
    idx_bufs = (idx_b0, idx_b1, idx_b2, idx_b3)
    n_streams = len(idx_bufs)
    SPB = KC // n_streams

    wid = lax.axis_index("s") * _NC + lax.axis_index("c")
    base_pt = wid * (BN // _NW)
    boff = (base_pt // N) * N

    io = lax.iota(jnp.int32, _L)
    io_k = io * k
    cols = [io + cc * _L for cc in range(C // _L)]

    def chunk_body(ch, _):
        p0 = base_pt + ch * CH
        pltpu.sync_copy(idx_hbm.at[pl.ds(p0 * k, KC)], idx_raw)
        pltpu.sync_copy(pts_hbm.at[pl.ds(p0, CH)], xv)
        for s in range(KC // _L):
            v = idx_raw[pl.ds(s * _L, _L)] + boff
            idx_bufs[(s * _L) // SPB][pl.ds((s * _L) % SPB, _L)] = v
        copies = [
            pltpu.async_copy(pts_hbm.at[idx_bufs[t]],
                             rows.at[pl.ds(t * SPB, SPB)], sem)
            for t in range(n_streams)
        ]
        for c_ in copies:
            c_.wait()

        def point_body(p, _):
            xr = [xv[p, pl.ds(cc * _L, _L)] for cc in range(C // _L)]
            obase = p * OUTW
            for cc in range(C // _L):
                for j in range(k):
                    g = rows[p * k + j, pl.ds(cc * _L, _L)]
                    sidx = io_k + (obase + cc * _L * k + j)
                    plsc.store_scatter(out_v, [sidx], g - xr[cc])
                    plsc.store_scatter(out_v, [sidx + C * k], xr[cc])
            return ()

        lax.fori_loop(0, CH, point_body, (), unroll=False)
        pltpu.sync_copy(out_v, out_hbm.at[pl.ds(p0 * OUTW, CH * OUTW)])
        return ()

    lax.fori_loop(0, n_chunks, chunk_body, (), unroll=False)


def kernel(points, idx):
    B, N, C = points.shape
    k = idx.shape[2]
    BN = B * N
    CH = 16

    pts_flat = points.reshape(BN, C)
    idx_flat = idx.reshape(BN * k)

    mesh = plsc.VectorSubcoreMesh(core_axis_name="c", subcore_axis_name="s")
    body = functools.partial(_sc_body, CH, BN, N, C, k)
    sc_fn = pl.kernel(
        body,
        out_type=jax.ShapeDtypeStruct((BN * 2 * C * k,), jnp.float32),
        mesh=mesh,
        compiler_params=pltpu.CompilerParams(needs_layout_passes=False,
                                              use_tc_tiling_on_sc=False),
        scratch_types=[
            pltpu.VMEM((k * CH,), jnp.int32),
            pltpu.VMEM((k * CH // 4,), jnp.int32),
            pltpu.VMEM((k * CH // 4,), jnp.int32),
            pltpu.VMEM((k * CH // 4,), jnp.int32),
            pltpu.VMEM((k * CH // 4,), jnp.int32),
            pltpu.VMEM((CH, C), jnp.float32),
            pltpu.VMEM((k * CH, C), jnp.float32),
            pltpu.VMEM((CH * 2 * C * k,), jnp.float32),
            pltpu.SemaphoreType.DMA,
        ],
    )
    out = sc_fn(pts_flat, idx_flat)
    return out.reshape(B, N, 2 * C, k)
